# Initial kernel scaffold; baseline (speedup 1.0000x reference)
import jax
import jax.numpy as jnp
from jax.experimental import pallas as pl


def kernel(x, stem_w, stem_gamma, stem_beta, stem_mean, stem_var, l1_0_conv1_w, l1_0_conv1_gamma, l1_0_conv1_beta, l1_0_conv1_mean, l1_0_conv1_var, l1_0_conv2_w, l1_0_conv2_gamma, l1_0_conv2_beta, l1_0_conv2_mean, l1_0_conv2_var, l1_1_conv1_w, l1_1_conv1_gamma, l1_1_conv1_beta, l1_1_conv1_mean, l1_1_conv1_var, l1_1_conv2_w, l1_1_conv2_gamma, l1_1_conv2_beta, l1_1_conv2_mean, l1_1_conv2_var, l2_0_conv1_w, l2_0_conv1_gamma, l2_0_conv1_beta, l2_0_conv1_mean, l2_0_conv1_var, l2_0_conv2_w, l2_0_conv2_gamma, l2_0_conv2_beta, l2_0_conv2_mean, l2_0_conv2_var, l2_0_ds_w, l2_0_ds_gamma, l2_0_ds_beta, l2_0_ds_mean, l2_0_ds_var, l2_1_conv1_w, l2_1_conv1_gamma, l2_1_conv1_beta, l2_1_conv1_mean, l2_1_conv1_var, l2_1_conv2_w, l2_1_conv2_gamma, l2_1_conv2_beta, l2_1_conv2_mean, l2_1_conv2_var, l3_0_conv1_w, l3_0_conv1_gamma, l3_0_conv1_beta, l3_0_conv1_mean, l3_0_conv1_var, l3_0_conv2_w, l3_0_conv2_gamma, l3_0_conv2_beta, l3_0_conv2_mean, l3_0_conv2_var, l3_0_ds_w, l3_0_ds_gamma, l3_0_ds_beta, l3_0_ds_mean, l3_0_ds_var, l3_1_conv1_w, l3_1_conv1_gamma, l3_1_conv1_beta, l3_1_conv1_mean, l3_1_conv1_var, l3_1_conv2_w, l3_1_conv2_gamma, l3_1_conv2_beta, l3_1_conv2_mean, l3_1_conv2_var):
    raise NotImplementedError("write your pallas kernel here")


# speedup vs baseline: 4.6987x; 4.6987x over previous
import functools

import jax
import jax.numpy as jnp
from jax.experimental import pallas as pl
from jax.experimental.pallas import tpu as pltpu

BN_EPS = 1e-5
BF16 = jnp.bfloat16
F32 = jnp.float32


def _fold_bn(g, b, m, v):
    s = g / jnp.sqrt(v + BN_EPS)
    return (s.reshape(1, -1).astype(F32), (b - m * s).reshape(1, -1).astype(F32))


def _wk_s1(w):
    n, c, _, _ = w.shape
    return jnp.transpose(w, (2, 1, 3, 0)).reshape(3 * c, 3 * n).astype(BF16)


def _wk_taps(w):
    n, c, _, _ = w.shape
    return jnp.transpose(w, (2, 3, 1, 0)).reshape(9, c, n).astype(BF16)


def _conv3_chunk(src, r0, tr, W, C, wmat):
    Wp = src.shape[1]
    xs = src[r0:r0 + tr + 2, :, :]
    lhs = jnp.concatenate([xs[0:tr], xs[1:tr + 1], xs[2:tr + 2]], axis=-1)
    n3 = wmat.shape[1]
    N = n3 // 3
    z = jnp.dot(lhs.reshape(tr * Wp, 3 * C), wmat,
                preferred_element_type=F32).reshape(tr, Wp, n3)
    return (z[:, 0:W, 0:N] + z[:, 1:1 + W, N:2 * N] + z[:, 2:2 + W, 2 * N:3 * N])


def _block_s1_body(xp_ref, w1_ref, s1_ref, b1_ref, w2_ref, s2_ref, b2_ref,
                   o_ref, h_ref, *, H, W, C, tr, pad_out):
    bb = xp_ref.shape[0]
    if pad_out:
        o_ref[...] = jnp.zeros(o_ref.shape, o_ref.dtype)
    h_ref[...] = jnp.zeros(h_ref.shape, h_ref.dtype)
    s1 = s1_ref[...]
    b1 = b1_ref[...]
    w1 = w1_ref[...]
    for b in range(bb):
        for r0 in range(0, H, tr):
            acc = _conv3_chunk(xp_ref[b], r0, tr, W, C, w1)
            hc = jnp.maximum(acc * s1 + b1, 0.0).astype(BF16)
            h_ref[b, 1 + r0:1 + r0 + tr, 1:1 + W, :] = hc
    s2 = s2_ref[...]
    b2 = b2_ref[...]
    w2 = w2_ref[...]
    oy = 1 if pad_out else 0
    for b in range(bb):
        for r0 in range(0, H, tr):
            acc = _conv3_chunk(h_ref[b], r0, tr, W, C, w2)
            res = xp_ref[b, 1 + r0:1 + r0 + tr, 1:1 + W, :].astype(F32)
            out = jnp.maximum(acc * s2 + b2 + res, 0.0)
            o_ref[b, oy + r0:oy + r0 + tr, oy:oy + W, :] = out.astype(o_ref.dtype)


def _block_s1(xp, p1, p2, H, W, C, tr, bb, pad_out, out_dtype):
    B = xp.shape[0]
    Wp = xp.shape[2]
    s1, b1 = _fold_bn(*p1[1:])
    s2, b2 = _fold_bn(*p2[1:])
    w1k = _wk_s1(p1[0])
    w2k = _wk_s1(p2[0])
    out_shape = (B, H + 2, Wp, C) if pad_out else (B, H, W, C)
    body = functools.partial(_block_s1_body, H=H, W=W, C=C, tr=tr,
                             pad_out=pad_out)
    return pl.pallas_call(
        body,
        out_shape=jax.ShapeDtypeStruct(out_shape, out_dtype),
        grid=(B // bb,),
        in_specs=[
            pl.BlockSpec((bb,) + xp.shape[1:], lambda i: (i, 0, 0, 0)),
            pl.BlockSpec(w1k.shape, lambda i: (0, 0)),
            pl.BlockSpec(s1.shape, lambda i: (0, 0)),
            pl.BlockSpec(b1.shape, lambda i: (0, 0)),
            pl.BlockSpec(w2k.shape, lambda i: (0, 0)),
            pl.BlockSpec(s2.shape, lambda i: (0, 0)),
            pl.BlockSpec(b2.shape, lambda i: (0, 0)),
        ],
        out_specs=pl.BlockSpec((bb,) + out_shape[1:], lambda i: (i, 0, 0, 0)),
        scratch_shapes=[pltpu.VMEM((bb, H + 2, Wp, C), BF16)],
        compiler_params=pltpu.CompilerParams(dimension_semantics=("parallel",)),
    )(xp, w1k, s1, b1, w2k, s2, b2)


def _block_ds_body(p_ref, w1_ref, s1_ref, b1_ref, w2_ref, s2_ref, b2_ref,
                   wd_ref, sd_ref, bd_ref, o_ref, h_ref, *,
                   Ho, Wo, Ci, Co, tr, pad_out):
    bb = p_ref.shape[0]
    Wh = p_ref.shape[4]
    if pad_out:
        o_ref[...] = jnp.zeros(o_ref.shape, o_ref.dtype)
    h_ref[...] = jnp.zeros(h_ref.shape, h_ref.dtype)
    s1 = s1_ref[...]
    b1 = b1_ref[...]
    for b in range(bb):
        for r0 in range(0, Ho, tr):
            acc = None
            for ky in range(3):
                a, dy = ky % 2, ky // 2
                for kx in range(3):
                    pb, dx = kx % 2, kx // 2
                    lhs = p_ref[b, a, pb, r0 + dy:r0 + dy + tr, :, :]
                    z = jnp.dot(lhs.reshape(tr * Wh, Ci), w1_ref[ky * 3 + kx],
                                preferred_element_type=F32).reshape(tr, Wh, Co)
                    c = z[:, dx:dx + Wo, :]
                    acc = c if acc is None else acc + c
            hc = jnp.maximum(acc * s1 + b1, 0.0).astype(BF16)
            h_ref[b, 1 + r0:1 + r0 + tr, 1:1 + Wo, :] = hc
    s2 = s2_ref[...]
    b2 = b2_ref[...]
    sd = sd_ref[...]
    bd = bd_ref[...]
    wd = wd_ref[...]
    w2 = w2_ref[...]
    oy = 1 if pad_out else 0
    for b in range(bb):
        for r0 in range(0, Ho, tr):
            acc = _conv3_chunk(h_ref[b], r0, tr, Wo, Co, w2)
            ld = p_ref[b, 1, 1, r0:r0 + tr, :, :]
            zd = jnp.dot(ld.reshape(tr * Wh, Ci), wd,
                         preferred_element_type=F32).reshape(tr, Wh, Co)
            res = zd[:, 0:Wo, :] * sd + bd
            out = jnp.maximum(acc * s2 + b2 + res, 0.0)
            o_ref[b, oy + r0:oy + r0 + tr, oy:oy + Wo, :] = out.astype(o_ref.dtype)


def _block_ds(ph, p1, p2, pd, Ho, Wo, Ci, Co, tr, bb, pad_out, out_dtype):
    B = ph.shape[0]
    s1, b1 = _fold_bn(*p1[1:])
    s2, b2 = _fold_bn(*p2[1:])
    sd, bd = _fold_bn(*pd[1:])
    w1k = _wk_taps(p1[0])
    w2k = _wk_s1(p2[0])
    wdk = jnp.transpose(pd[0].reshape(Co, Ci), (1, 0)).astype(BF16)
    Wp2 = ph.shape[4]
    out_shape = (B, Ho + 2, Wp2, Co) if pad_out else (B, Ho, Wo, Co)
    body = functools.partial(_block_ds_body, Ho=Ho, Wo=Wo, Ci=Ci, Co=Co,
                             tr=tr, pad_out=pad_out)
    return pl.pallas_call(
        body,
        out_shape=jax.ShapeDtypeStruct(out_shape, out_dtype),
        grid=(B // bb,),
        in_specs=[
            pl.BlockSpec((bb,) + ph.shape[1:], lambda i: (i, 0, 0, 0, 0, 0)),
            pl.BlockSpec(w1k.shape, lambda i: (0, 0, 0)),
            pl.BlockSpec(s1.shape, lambda i: (0, 0)),
            pl.BlockSpec(b1.shape, lambda i: (0, 0)),
            pl.BlockSpec(w2k.shape, lambda i: (0, 0)),
            pl.BlockSpec(s2.shape, lambda i: (0, 0)),
            pl.BlockSpec(b2.shape, lambda i: (0, 0)),
            pl.BlockSpec(wdk.shape, lambda i: (0, 0)),
            pl.BlockSpec(sd.shape, lambda i: (0, 0)),
            pl.BlockSpec(bd.shape, lambda i: (0, 0)),
        ],
        out_specs=pl.BlockSpec((bb,) + out_shape[1:], lambda i: (i, 0, 0, 0)),
        scratch_shapes=[pltpu.VMEM((bb, Ho + 2, Wp2, Co), BF16)],
        compiler_params=pltpu.CompilerParams(dimension_semantics=("parallel",)),
    )(ph, w1k, s1, b1, w2k, s2, b2, wdk, sd, bd)


def _stem_body(p_ref, w_ref, s_ref, b_ref, o_ref):
    z = jnp.dot(p_ref[...], w_ref[...], preferred_element_type=F32)
    o_ref[...] = jnp.maximum(z * s_ref[...] + b_ref[...], 0.0).astype(BF16)


def _pool_body(q_ref, o_ref, *, tr):
    bb = q_ref.shape[0]
    o_ref[...] = jnp.zeros(o_ref.shape, o_ref.dtype)
    for b in range(bb):
        for r0 in range(0, 56, tr):
            acc = None
            for ky in range(3):
                a, dy = ky % 2, ky // 2
                for kx in range(3):
                    pb, dx = kx % 2, kx // 2
                    t = q_ref[b, a, pb, r0 + dy:r0 + dy + tr, dx:dx + 56, :]
                    acc = t if acc is None else jnp.maximum(acc, t)
            o_ref[b, 1 + r0:1 + r0 + tr, 1:57, :] = acc


def _mk_phases(x, hq):
    B, H, W, C = x.shape
    xq = jnp.pad(x, ((0, 0), (1, 2 * hq - H - 1), (1, 2 * hq - W - 1), (0, 0)))
    return xq.reshape(B, hq, 2, hq, 2, C).transpose(0, 2, 4, 1, 3, 5)


def kernel(x, stem_w, stem_gamma, stem_beta, stem_mean, stem_var, l1_0_conv1_w, l1_0_conv1_gamma, l1_0_conv1_beta, l1_0_conv1_mean, l1_0_conv1_var, l1_0_conv2_w, l1_0_conv2_gamma, l1_0_conv2_beta, l1_0_conv2_mean, l1_0_conv2_var, l1_1_conv1_w, l1_1_conv1_gamma, l1_1_conv1_beta, l1_1_conv1_mean, l1_1_conv1_var, l1_1_conv2_w, l1_1_conv2_gamma, l1_1_conv2_beta, l1_1_conv2_mean, l1_1_conv2_var, l2_0_conv1_w, l2_0_conv1_gamma, l2_0_conv1_beta, l2_0_conv1_mean, l2_0_conv1_var, l2_0_conv2_w, l2_0_conv2_gamma, l2_0_conv2_beta, l2_0_conv2_mean, l2_0_conv2_var, l2_0_ds_w, l2_0_ds_gamma, l2_0_ds_beta, l2_0_ds_mean, l2_0_ds_var, l2_1_conv1_w, l2_1_conv1_gamma, l2_1_conv1_beta, l2_1_conv1_mean, l2_1_conv1_var, l2_1_conv2_w, l2_1_conv2_gamma, l2_1_conv2_beta, l2_1_conv2_mean, l2_1_conv2_var, l3_0_conv1_w, l3_0_conv1_gamma, l3_0_conv1_beta, l3_0_conv1_mean, l3_0_conv1_var, l3_0_conv2_w, l3_0_conv2_gamma, l3_0_conv2_beta, l3_0_conv2_mean, l3_0_conv2_var, l3_0_ds_w, l3_0_ds_gamma, l3_0_ds_beta, l3_0_ds_mean, l3_0_ds_var, l3_1_conv1_w, l3_1_conv1_gamma, l3_1_conv1_beta, l3_1_conv1_mean, l3_1_conv1_var, l3_1_conv2_w, l3_1_conv2_gamma, l3_1_conv2_beta, l3_1_conv2_mean, l3_1_conv2_var):
    B = x.shape[0]

    xb = jnp.transpose(x, (0, 2, 3, 1)).astype(BF16)
    xpad = jnp.pad(xb, ((0, 0), (3, 3), (3, 3), (0, 0)))
    cols = []
    for ky in range(7):
        for kx in range(7):
            cols.append(xpad[:, ky:ky + 223:2, kx:kx + 223:2, :])
    patches = jnp.concatenate(cols, axis=-1).reshape(B * 112 * 112, 147)
    w_mat = jnp.transpose(stem_w, (2, 3, 1, 0)).reshape(147, 64).astype(BF16)
    ss, sb = _fold_bn(stem_gamma, stem_beta, stem_mean, stem_var)
    M = B * 112 * 112
    tm = 14336
    stem_out = pl.pallas_call(
        _stem_body,
        out_shape=jax.ShapeDtypeStruct((M, 64), BF16),
        grid=(M // tm,),
        in_specs=[
            pl.BlockSpec((tm, 147), lambda i: (i, 0)),
            pl.BlockSpec((147, 64), lambda i: (0, 0)),
            pl.BlockSpec((1, 64), lambda i: (0, 0)),
            pl.BlockSpec((1, 64), lambda i: (0, 0)),
        ],
        out_specs=pl.BlockSpec((tm, 64), lambda i: (i, 0)),
        compiler_params=pltpu.CompilerParams(dimension_semantics=("parallel",)),
    )(patches, w_mat, ss, sb)

    s4 = stem_out.reshape(B, 112, 112, 64)
    sq = jnp.pad(s4, ((0, 0), (1, 1), (1, 1), (0, 0)))
    q = sq.reshape(B, 57, 2, 57, 2, 64).transpose(0, 2, 4, 1, 3, 5)
    pb = 4
    x1 = pl.pallas_call(
        functools.partial(_pool_body, tr=8),
        out_shape=jax.ShapeDtypeStruct((B, 58, 64, 64), BF16),
        grid=(B // pb,),
        in_specs=[pl.BlockSpec((pb, 2, 2, 57, 57, 64),
                               lambda i: (i, 0, 0, 0, 0, 0))],
        out_specs=pl.BlockSpec((pb, 58, 64, 64), lambda i: (i, 0, 0, 0)),
        compiler_params=pltpu.CompilerParams(dimension_semantics=("parallel",)),
    )(q)

    l10 = ((l1_0_conv1_w, l1_0_conv1_gamma, l1_0_conv1_beta, l1_0_conv1_mean, l1_0_conv1_var),
           (l1_0_conv2_w, l1_0_conv2_gamma, l1_0_conv2_beta, l1_0_conv2_mean, l1_0_conv2_var))
    l11 = ((l1_1_conv1_w, l1_1_conv1_gamma, l1_1_conv1_beta, l1_1_conv1_mean, l1_1_conv1_var),
           (l1_1_conv2_w, l1_1_conv2_gamma, l1_1_conv2_beta, l1_1_conv2_mean, l1_1_conv2_var))
    x1 = _block_s1(x1, *l10, H=56, W=56, C=64, tr=14, bb=1, pad_out=True,
                   out_dtype=BF16)
    x1 = _block_s1(x1, *l11, H=56, W=56, C=64, tr=14, bb=1, pad_out=False,
                   out_dtype=BF16)

    q2 = _mk_phases(x1, 32)
    l20 = ((l2_0_conv1_w, l2_0_conv1_gamma, l2_0_conv1_beta, l2_0_conv1_mean, l2_0_conv1_var),
           (l2_0_conv2_w, l2_0_conv2_gamma, l2_0_conv2_beta, l2_0_conv2_mean, l2_0_conv2_var),
           (l2_0_ds_w, l2_0_ds_gamma, l2_0_ds_beta, l2_0_ds_mean, l2_0_ds_var))
    l21 = ((l2_1_conv1_w, l2_1_conv1_gamma, l2_1_conv1_beta, l2_1_conv1_mean, l2_1_conv1_var),
           (l2_1_conv2_w, l2_1_conv2_gamma, l2_1_conv2_beta, l2_1_conv2_mean, l2_1_conv2_var))
    x2 = _block_ds(q2, *l20, Ho=28, Wo=28, Ci=64, Co=128, tr=14, bb=4,
                   pad_out=True, out_dtype=BF16)
    x2 = _block_s1(x2, *l21, H=28, W=28, C=128, tr=14, bb=4, pad_out=False,
                   out_dtype=BF16)

    q3 = _mk_phases(x2, 16)
    l30 = ((l3_0_conv1_w, l3_0_conv1_gamma, l3_0_conv1_beta, l3_0_conv1_mean, l3_0_conv1_var),
           (l3_0_conv2_w, l3_0_conv2_gamma, l3_0_conv2_beta, l3_0_conv2_mean, l3_0_conv2_var),
           (l3_0_ds_w, l3_0_ds_gamma, l3_0_ds_beta, l3_0_ds_mean, l3_0_ds_var))
    l31 = ((l3_1_conv1_w, l3_1_conv1_gamma, l3_1_conv1_beta, l3_1_conv1_mean, l3_1_conv1_var),
           (l3_1_conv2_w, l3_1_conv2_gamma, l3_1_conv2_beta, l3_1_conv2_mean, l3_1_conv2_var))
    x3 = _block_ds(q3, *l30, Ho=14, Wo=14, Ci=128, Co=256, tr=14, bb=8,
                   pad_out=True, out_dtype=BF16)
    x3 = _block_s1(x3, *l31, H=14, W=14, C=256, tr=14, bb=8, pad_out=False,
                   out_dtype=F32)

    return jnp.transpose(x3, (0, 3, 1, 2))


# speedup vs baseline: 9.4051x; 2.0017x over previous
import functools

import jax
import jax.numpy as jnp
from jax.experimental import pallas as pl
from jax.experimental.pallas import tpu as pltpu

BN_EPS = 1e-5
BF16 = jnp.bfloat16
F32 = jnp.float32


def _fold_bn(g, b, m, v):
    s = g / jnp.sqrt(v + BN_EPS)
    return (s.reshape(1, -1).astype(F32), (b - m * s).reshape(1, -1).astype(F32))


def _wk_s1(w):
    n, c, _, _ = w.shape
    return jnp.transpose(w, (2, 1, 3, 0)).reshape(3 * c, 3 * n).astype(BF16)


def _wk_taps(w):
    n, c, _, _ = w.shape
    return jnp.transpose(w, (2, 3, 1, 0)).reshape(9, c, n).astype(BF16)


def _conv3_chunk(src, r0, tr, W, C, wmat):
    Wp = src.shape[1]
    xs = src[r0:r0 + tr + 2, :, :]
    lhs = jnp.concatenate([xs[0:tr], xs[1:tr + 1], xs[2:tr + 2]], axis=-1)
    n3 = wmat.shape[1]
    N = n3 // 3
    z = jnp.dot(lhs.reshape(tr * Wp, 3 * C), wmat,
                preferred_element_type=F32).reshape(tr, Wp, n3)
    return (z[:, 0:W, 0:N] + z[:, 1:1 + W, N:2 * N] + z[:, 2:2 + W, 2 * N:3 * N])


def _block_s1_body(xp_ref, w1_ref, s1_ref, b1_ref, w2_ref, s2_ref, b2_ref,
                   o_ref, h_ref, *, H, W, C, tr, pad_out):
    bb = xp_ref.shape[0]
    if pad_out:
        o_ref[...] = jnp.zeros(o_ref.shape, o_ref.dtype)
    h_ref[...] = jnp.zeros(h_ref.shape, h_ref.dtype)
    s1 = s1_ref[...]
    b1 = b1_ref[...]
    w1 = w1_ref[...]
    for b in range(bb):
        for r0 in range(0, H, tr):
            acc = _conv3_chunk(xp_ref[b], r0, tr, W, C, w1)
            hc = jnp.maximum(acc * s1 + b1, 0.0).astype(BF16)
            h_ref[b, 1 + r0:1 + r0 + tr, 1:1 + W, :] = hc
    s2 = s2_ref[...]
    b2 = b2_ref[...]
    w2 = w2_ref[...]
    oy = 1 if pad_out else 0
    for b in range(bb):
        for r0 in range(0, H, tr):
            acc = _conv3_chunk(h_ref[b], r0, tr, W, C, w2)
            res = xp_ref[b, 1 + r0:1 + r0 + tr, 1:1 + W, :].astype(F32)
            out = jnp.maximum(acc * s2 + b2 + res, 0.0)
            o_ref[b, oy + r0:oy + r0 + tr, oy:oy + W, :] = out.astype(o_ref.dtype)


def _block_s1(xp, p1, p2, H, W, C, tr, bb, pad_out, out_dtype):
    B = xp.shape[0]
    Wp = xp.shape[2]
    s1, b1 = _fold_bn(*p1[1:])
    s2, b2 = _fold_bn(*p2[1:])
    w1k = _wk_s1(p1[0])
    w2k = _wk_s1(p2[0])
    out_shape = (B, H + 2, Wp, C) if pad_out else (B, H, W, C)
    body = functools.partial(_block_s1_body, H=H, W=W, C=C, tr=tr,
                             pad_out=pad_out)
    return pl.pallas_call(
        body,
        out_shape=jax.ShapeDtypeStruct(out_shape, out_dtype),
        grid=(B // bb,),
        in_specs=[
            pl.BlockSpec((bb,) + xp.shape[1:], lambda i: (i, 0, 0, 0)),
            pl.BlockSpec(w1k.shape, lambda i: (0, 0)),
            pl.BlockSpec(s1.shape, lambda i: (0, 0)),
            pl.BlockSpec(b1.shape, lambda i: (0, 0)),
            pl.BlockSpec(w2k.shape, lambda i: (0, 0)),
            pl.BlockSpec(s2.shape, lambda i: (0, 0)),
            pl.BlockSpec(b2.shape, lambda i: (0, 0)),
        ],
        out_specs=pl.BlockSpec((bb,) + out_shape[1:], lambda i: (i, 0, 0, 0)),
        scratch_shapes=[pltpu.VMEM((bb, H + 2, Wp, C), BF16)],
        compiler_params=pltpu.CompilerParams(dimension_semantics=("parallel",)),
    )(xp, w1k, s1, b1, w2k, s2, b2)


def _block_ds_body(p_ref, w1_ref, s1_ref, b1_ref, w2_ref, s2_ref, b2_ref,
                   wd_ref, sd_ref, bd_ref, o_ref, h_ref, *,
                   Ho, Wo, Ci, Co, tr, pad_out):
    bb = p_ref.shape[0]
    Wh = p_ref.shape[4]
    if pad_out:
        o_ref[...] = jnp.zeros(o_ref.shape, o_ref.dtype)
    h_ref[...] = jnp.zeros(h_ref.shape, h_ref.dtype)
    s1 = s1_ref[...]
    b1 = b1_ref[...]
    for b in range(bb):
        for r0 in range(0, Ho, tr):
            acc = None
            for ky in range(3):
                a, dy = ky % 2, ky // 2
                for kx in range(3):
                    pb, dx = kx % 2, kx // 2
                    lhs = p_ref[b, a, pb, r0 + dy:r0 + dy + tr, :, :]
                    z = jnp.dot(lhs.reshape(tr * Wh, Ci), w1_ref[ky * 3 + kx],
                                preferred_element_type=F32).reshape(tr, Wh, Co)
                    c = z[:, dx:dx + Wo, :]
                    acc = c if acc is None else acc + c
            hc = jnp.maximum(acc * s1 + b1, 0.0).astype(BF16)
            h_ref[b, 1 + r0:1 + r0 + tr, 1:1 + Wo, :] = hc
    s2 = s2_ref[...]
    b2 = b2_ref[...]
    sd = sd_ref[...]
    bd = bd_ref[...]
    wd = wd_ref[...]
    w2 = w2_ref[...]
    oy = 1 if pad_out else 0
    for b in range(bb):
        for r0 in range(0, Ho, tr):
            acc = _conv3_chunk(h_ref[b], r0, tr, Wo, Co, w2)
            ld = p_ref[b, 1, 1, r0:r0 + tr, :, :]
            zd = jnp.dot(ld.reshape(tr * Wh, Ci), wd,
                         preferred_element_type=F32).reshape(tr, Wh, Co)
            res = zd[:, 0:Wo, :] * sd + bd
            out = jnp.maximum(acc * s2 + b2 + res, 0.0)
            o_ref[b, oy + r0:oy + r0 + tr, oy:oy + Wo, :] = out.astype(o_ref.dtype)


def _block_ds(ph, p1, p2, pd, Ho, Wo, Ci, Co, tr, bb, pad_out, out_dtype):
    B = ph.shape[0]
    s1, b1 = _fold_bn(*p1[1:])
    s2, b2 = _fold_bn(*p2[1:])
    sd, bd = _fold_bn(*pd[1:])
    w1k = _wk_taps(p1[0])
    w2k = _wk_s1(p2[0])
    wdk = jnp.transpose(pd[0].reshape(Co, Ci), (1, 0)).astype(BF16)
    Wp2 = ph.shape[4]
    out_shape = (B, Ho + 2, Wp2, Co) if pad_out else (B, Ho, Wo, Co)
    body = functools.partial(_block_ds_body, Ho=Ho, Wo=Wo, Ci=Ci, Co=Co,
                             tr=tr, pad_out=pad_out)
    return pl.pallas_call(
        body,
        out_shape=jax.ShapeDtypeStruct(out_shape, out_dtype),
        grid=(B // bb,),
        in_specs=[
            pl.BlockSpec((bb,) + ph.shape[1:], lambda i: (i, 0, 0, 0, 0, 0)),
            pl.BlockSpec(w1k.shape, lambda i: (0, 0, 0)),
            pl.BlockSpec(s1.shape, lambda i: (0, 0)),
            pl.BlockSpec(b1.shape, lambda i: (0, 0)),
            pl.BlockSpec(w2k.shape, lambda i: (0, 0)),
            pl.BlockSpec(s2.shape, lambda i: (0, 0)),
            pl.BlockSpec(b2.shape, lambda i: (0, 0)),
            pl.BlockSpec(wdk.shape, lambda i: (0, 0)),
            pl.BlockSpec(sd.shape, lambda i: (0, 0)),
            pl.BlockSpec(bd.shape, lambda i: (0, 0)),
        ],
        out_specs=pl.BlockSpec((bb,) + out_shape[1:], lambda i: (i, 0, 0, 0)),
        scratch_shapes=[pltpu.VMEM((bb, Ho + 2, Wp2, Co), BF16)],
        compiler_params=pltpu.CompilerParams(dimension_semantics=("parallel",)),
    )(ph, w1k, s1, b1, w2k, s2, b2, wdk, sd, bd)


def _stem_pool_body(p_ref, w_ref, s_ref, b_ref, o_ref, cp_ref):
    w = w_ref[...]
    s = s_ref[...]
    bsh = b_ref[...]
    o_ref[...] = jnp.zeros(o_ref.shape, o_ref.dtype)
    for c in range(8):
        lhs = p_ref[0, 14 * c:14 * c + 14, :, :].reshape(14 * 112, 192)
        z = jnp.dot(lhs, w, preferred_element_type=F32).reshape(14, 112, 64)
        z = jnp.maximum(z * s + bsh, 0.0)
        rs = z.reshape(14, 56, 2, 64)
        m1 = jnp.maximum(rs[:, :, 0, :], rs[:, :, 1, :])
        odd = rs[:, :, 1, :]
        o1 = jnp.concatenate([jnp.zeros((14, 1, 64), F32), odd[:, 0:55, :]],
                             axis=1)
        cp_ref[7 * c:7 * c + 7] = jnp.maximum(m1, o1).reshape(7, 2, 56, 64)
    a = cp_ref[:, 0]
    bq = cp_ref[:, 1]
    bshift = jnp.concatenate([jnp.zeros((1, 56, 64), F32), bq[0:55]], axis=0)
    pooled = jnp.maximum(jnp.maximum(a, bq), bshift)
    o_ref[0, 1:57, 1:57, :] = pooled.astype(BF16)


def _mk_phases(x, hq):
    B, H, W, C = x.shape
    xq = jnp.pad(x, ((0, 0), (1, 2 * hq - H - 1), (1, 2 * hq - W - 1), (0, 0)))
    return xq.reshape(B, hq, 2, hq, 2, C).transpose(0, 2, 4, 1, 3, 5)


def kernel(x, stem_w, stem_gamma, stem_beta, stem_mean, stem_var, l1_0_conv1_w, l1_0_conv1_gamma, l1_0_conv1_beta, l1_0_conv1_mean, l1_0_conv1_var, l1_0_conv2_w, l1_0_conv2_gamma, l1_0_conv2_beta, l1_0_conv2_mean, l1_0_conv2_var, l1_1_conv1_w, l1_1_conv1_gamma, l1_1_conv1_beta, l1_1_conv1_mean, l1_1_conv1_var, l1_1_conv2_w, l1_1_conv2_gamma, l1_1_conv2_beta, l1_1_conv2_mean, l1_1_conv2_var, l2_0_conv1_w, l2_0_conv1_gamma, l2_0_conv1_beta, l2_0_conv1_mean, l2_0_conv1_var, l2_0_conv2_w, l2_0_conv2_gamma, l2_0_conv2_beta, l2_0_conv2_mean, l2_0_conv2_var, l2_0_ds_w, l2_0_ds_gamma, l2_0_ds_beta, l2_0_ds_mean, l2_0_ds_var, l2_1_conv1_w, l2_1_conv1_gamma, l2_1_conv1_beta, l2_1_conv1_mean, l2_1_conv1_var, l2_1_conv2_w, l2_1_conv2_gamma, l2_1_conv2_beta, l2_1_conv2_mean, l2_1_conv2_var, l3_0_conv1_w, l3_0_conv1_gamma, l3_0_conv1_beta, l3_0_conv1_mean, l3_0_conv1_var, l3_0_conv2_w, l3_0_conv2_gamma, l3_0_conv2_beta, l3_0_conv2_mean, l3_0_conv2_var, l3_0_ds_w, l3_0_ds_gamma, l3_0_ds_beta, l3_0_ds_mean, l3_0_ds_var, l3_1_conv1_w, l3_1_conv1_gamma, l3_1_conv1_beta, l3_1_conv1_mean, l3_1_conv1_var, l3_1_conv2_w, l3_1_conv2_gamma, l3_1_conv2_beta, l3_1_conv2_mean, l3_1_conv2_var):
    B = x.shape[0]

    xb = x.astype(BF16)
    xpad = jnp.pad(xb, ((0, 0), (0, 0), (3, 3), (3, 5)))
    xph = xpad.reshape(B, 3, 115, 2, 116, 2).transpose(0, 2, 4, 3, 5, 1)
    xph = xph.reshape(B, 115, 116, 12)
    pieces = []
    for dy in range(4):
        for dx in range(4):
            pieces.append(xph[:, dy:dy + 112, dx:dx + 112, :])
    patches = jnp.concatenate(pieces, axis=-1)
    wp = jnp.pad(stem_w, ((0, 0), (0, 0), (0, 1), (0, 1)))
    w_mat = wp.reshape(64, 3, 4, 2, 4, 2).transpose(2, 4, 3, 5, 1, 0)
    w_mat = w_mat.reshape(192, 64).astype(BF16)
    ss, sb = _fold_bn(stem_gamma, stem_beta, stem_mean, stem_var)
    x1 = pl.pallas_call(
        _stem_pool_body,
        out_shape=jax.ShapeDtypeStruct((B, 58, 64, 64), BF16),
        grid=(B,),
        in_specs=[
            pl.BlockSpec((1, 112, 112, 192), lambda i: (i, 0, 0, 0)),
            pl.BlockSpec((192, 64), lambda i: (0, 0)),
            pl.BlockSpec((1, 64), lambda i: (0, 0)),
            pl.BlockSpec((1, 64), lambda i: (0, 0)),
        ],
        out_specs=pl.BlockSpec((1, 58, 64, 64), lambda i: (i, 0, 0, 0)),
        scratch_shapes=[pltpu.VMEM((56, 2, 56, 64), F32)],
        compiler_params=pltpu.CompilerParams(dimension_semantics=("parallel",)),
    )(patches, w_mat, ss, sb)

    l10 = ((l1_0_conv1_w, l1_0_conv1_gamma, l1_0_conv1_beta, l1_0_conv1_mean, l1_0_conv1_var),
           (l1_0_conv2_w, l1_0_conv2_gamma, l1_0_conv2_beta, l1_0_conv2_mean, l1_0_conv2_var))
    l11 = ((l1_1_conv1_w, l1_1_conv1_gamma, l1_1_conv1_beta, l1_1_conv1_mean, l1_1_conv1_var),
           (l1_1_conv2_w, l1_1_conv2_gamma, l1_1_conv2_beta, l1_1_conv2_mean, l1_1_conv2_var))
    x1 = _block_s1(x1, *l10, H=56, W=56, C=64, tr=14, bb=1, pad_out=True,
                   out_dtype=BF16)
    x1 = _block_s1(x1, *l11, H=56, W=56, C=64, tr=14, bb=1, pad_out=False,
                   out_dtype=BF16)

    q2 = _mk_phases(x1, 32)
    l20 = ((l2_0_conv1_w, l2_0_conv1_gamma, l2_0_conv1_beta, l2_0_conv1_mean, l2_0_conv1_var),
           (l2_0_conv2_w, l2_0_conv2_gamma, l2_0_conv2_beta, l2_0_conv2_mean, l2_0_conv2_var),
           (l2_0_ds_w, l2_0_ds_gamma, l2_0_ds_beta, l2_0_ds_mean, l2_0_ds_var))
    l21 = ((l2_1_conv1_w, l2_1_conv1_gamma, l2_1_conv1_beta, l2_1_conv1_mean, l2_1_conv1_var),
           (l2_1_conv2_w, l2_1_conv2_gamma, l2_1_conv2_beta, l2_1_conv2_mean, l2_1_conv2_var))
    x2 = _block_ds(q2, *l20, Ho=28, Wo=28, Ci=64, Co=128, tr=14, bb=4,
                   pad_out=True, out_dtype=BF16)
    x2 = _block_s1(x2, *l21, H=28, W=28, C=128, tr=14, bb=4, pad_out=False,
                   out_dtype=BF16)

    q3 = _mk_phases(x2, 16)
    l30 = ((l3_0_conv1_w, l3_0_conv1_gamma, l3_0_conv1_beta, l3_0_conv1_mean, l3_0_conv1_var),
           (l3_0_conv2_w, l3_0_conv2_gamma, l3_0_conv2_beta, l3_0_conv2_mean, l3_0_conv2_var),
           (l3_0_ds_w, l3_0_ds_gamma, l3_0_ds_beta, l3_0_ds_mean, l3_0_ds_var))
    l31 = ((l3_1_conv1_w, l3_1_conv1_gamma, l3_1_conv1_beta, l3_1_conv1_mean, l3_1_conv1_var),
           (l3_1_conv2_w, l3_1_conv2_gamma, l3_1_conv2_beta, l3_1_conv2_mean, l3_1_conv2_var))
    x3 = _block_ds(q3, *l30, Ho=14, Wo=14, Ci=128, Co=256, tr=14, bb=8,
                   pad_out=True, out_dtype=BF16)
    x3 = _block_s1(x3, *l31, H=14, W=14, C=256, tr=14, bb=8, pad_out=False,
                   out_dtype=F32)

    return jnp.transpose(x3, (0, 3, 1, 2))


# speedup vs baseline: 16.6519x; 1.7705x over previous
import functools

import jax
import jax.numpy as jnp
from jax.experimental import pallas as pl
from jax.experimental.pallas import tpu as pltpu

BN_EPS = 1e-5
BF16 = jnp.bfloat16
F32 = jnp.float32


def _fold_bn(g, b, m, v):
    s = g / jnp.sqrt(v + BN_EPS)
    return (s.reshape(1, -1).astype(F32), (b - m * s).reshape(1, -1).astype(F32))


def _wk_s1(w):
    n, c, _, _ = w.shape
    return jnp.transpose(w, (2, 1, 3, 0)).reshape(3 * c, 3 * n).astype(BF16)


def _wk_taps(w):
    n, c, _, _ = w.shape
    return jnp.transpose(w, (2, 3, 1, 0)).reshape(9, c, n).astype(BF16)


def _conv3_chunk(src, r0, tr, W, C, wmat):
    Wp = src.shape[1]
    xs = src[r0:r0 + tr + 2, :, :]
    lhs = jnp.concatenate([xs[0:tr], xs[1:tr + 1], xs[2:tr + 2]], axis=-1)
    n3 = wmat.shape[1]
    N = n3 // 3
    z = jnp.dot(lhs.reshape(tr * Wp, 3 * C), wmat,
                preferred_element_type=F32).reshape(tr, Wp, n3)
    return (z[:, 0:W, 0:N] + z[:, 1:1 + W, N:2 * N] + z[:, 2:2 + W, 2 * N:3 * N])


def _block_s1_body(xp_ref, w1_ref, s1_ref, b1_ref, w2_ref, s2_ref, b2_ref,
                   o_ref, h_ref, *, H, W, C, tr, pad_out, phase_out):
    bb = xp_ref.shape[0]
    if pad_out or phase_out:
        o_ref[...] = jnp.zeros(o_ref.shape, o_ref.dtype)
    h_ref[...] = jnp.zeros(h_ref.shape, h_ref.dtype)
    s1 = s1_ref[...]
    b1 = b1_ref[...]
    w1 = w1_ref[...]
    for b in range(bb):
        for r0 in range(0, H, tr):
            acc = _conv3_chunk(xp_ref[b], r0, tr, W, C, w1)
            hc = jnp.maximum(acc * s1 + b1, 0.0).astype(BF16)
            h_ref[b, 1 + r0:1 + r0 + tr, 1:1 + W, :] = hc
    s2 = s2_ref[...]
    b2 = b2_ref[...]
    w2 = w2_ref[...]
    oy = 1 if pad_out else 0
    for b in range(bb):
        for r0 in range(0, H, tr):
            acc = _conv3_chunk(h_ref[b], r0, tr, W, C, w2)
            res = xp_ref[b, 1 + r0:1 + r0 + tr, 1:1 + W, :].astype(F32)
            out = jnp.maximum(acc * s2 + b2 + res, 0.0)
            if phase_out:
                pv = out.astype(o_ref.dtype)
                i0, h2, w2c = r0 // 2, tr // 2, W // 2
                vr = pv.reshape(h2, 2, W, C)
                for ai, ish in ((1, 0), (0, 1)):
                    g = vr[:, 1 - ai].reshape(h2, w2c, 2, C)
                    o_ref[b, ai, 1, i0 + ish:i0 + ish + h2, 0:w2c, :] = g[:, :, 0, :]
                    o_ref[b, ai, 0, i0 + ish:i0 + ish + h2, 1:w2c + 1, :] = g[:, :, 1, :]
            else:
                o_ref[b, oy + r0:oy + r0 + tr, oy:oy + W, :] = out.astype(o_ref.dtype)


def _block_s1(xp, p1, p2, H, W, C, tr, bb, pad_out, out_dtype,
              phase_out=False):
    B = xp.shape[0]
    Wp = xp.shape[2]
    s1, b1 = _fold_bn(*p1[1:])
    s2, b2 = _fold_bn(*p2[1:])
    w1k = _wk_s1(p1[0])
    w2k = _wk_s1(p2[0])
    if phase_out:
        hq = (((H + 3) // 2) + 7) // 8 * 8
        out_shape = (B, 2, 2, hq, hq, C)
    elif pad_out:
        out_shape = (B, H + 2, Wp, C)
    else:
        out_shape = (B, H, W, C)
    body = functools.partial(_block_s1_body, H=H, W=W, C=C, tr=tr,
                             pad_out=pad_out, phase_out=phase_out)
    return pl.pallas_call(
        body,
        out_shape=jax.ShapeDtypeStruct(out_shape, out_dtype),
        grid=(B // bb,),
        in_specs=[
            pl.BlockSpec((bb,) + xp.shape[1:], lambda i: (i, 0, 0, 0)),
            pl.BlockSpec(w1k.shape, lambda i: (0, 0)),
            pl.BlockSpec(s1.shape, lambda i: (0, 0)),
            pl.BlockSpec(b1.shape, lambda i: (0, 0)),
            pl.BlockSpec(w2k.shape, lambda i: (0, 0)),
            pl.BlockSpec(s2.shape, lambda i: (0, 0)),
            pl.BlockSpec(b2.shape, lambda i: (0, 0)),
        ],
        out_specs=pl.BlockSpec((bb,) + out_shape[1:],
                               (lambda i: (i, 0, 0, 0, 0, 0)) if phase_out
                               else (lambda i: (i, 0, 0, 0))),
        scratch_shapes=[pltpu.VMEM((bb, H + 2, Wp, C), BF16)],
        compiler_params=pltpu.CompilerParams(dimension_semantics=("parallel",)),
    )(xp, w1k, s1, b1, w2k, s2, b2)


def _block_ds_body(p_ref, w1_ref, s1_ref, b1_ref, w2_ref, s2_ref, b2_ref,
                   wd_ref, sd_ref, bd_ref, o_ref, h_ref, *,
                   Ho, Wo, Ci, Co, tr, pad_out):
    bb = p_ref.shape[0]
    Wh = p_ref.shape[4]
    if pad_out:
        o_ref[...] = jnp.zeros(o_ref.shape, o_ref.dtype)
    h_ref[...] = jnp.zeros(h_ref.shape, h_ref.dtype)
    s1 = s1_ref[...]
    b1 = b1_ref[...]
    for b in range(bb):
        for r0 in range(0, Ho, tr):
            acc = None
            for ky in range(3):
                a, dy = ky % 2, ky // 2
                for kx in range(3):
                    pb, dx = kx % 2, kx // 2
                    lhs = p_ref[b, a, pb, r0 + dy:r0 + dy + tr, :, :]
                    z = jnp.dot(lhs.reshape(tr * Wh, Ci), w1_ref[ky * 3 + kx],
                                preferred_element_type=F32).reshape(tr, Wh, Co)
                    c = z[:, dx:dx + Wo, :]
                    acc = c if acc is None else acc + c
            hc = jnp.maximum(acc * s1 + b1, 0.0).astype(BF16)
            h_ref[b, 1 + r0:1 + r0 + tr, 1:1 + Wo, :] = hc
    s2 = s2_ref[...]
    b2 = b2_ref[...]
    sd = sd_ref[...]
    bd = bd_ref[...]
    wd = wd_ref[...]
    w2 = w2_ref[...]
    oy = 1 if pad_out else 0
    for b in range(bb):
        for r0 in range(0, Ho, tr):
            acc = _conv3_chunk(h_ref[b], r0, tr, Wo, Co, w2)
            ld = p_ref[b, 1, 1, r0:r0 + tr, :, :]
            zd = jnp.dot(ld.reshape(tr * Wh, Ci), wd,
                         preferred_element_type=F32).reshape(tr, Wh, Co)
            res = zd[:, 0:Wo, :] * sd + bd
            out = jnp.maximum(acc * s2 + b2 + res, 0.0)
            o_ref[b, oy + r0:oy + r0 + tr, oy:oy + Wo, :] = out.astype(o_ref.dtype)


def _block_ds(ph, p1, p2, pd, Ho, Wo, Ci, Co, tr, bb, pad_out, out_dtype):
    B = ph.shape[0]
    s1, b1 = _fold_bn(*p1[1:])
    s2, b2 = _fold_bn(*p2[1:])
    sd, bd = _fold_bn(*pd[1:])
    w1k = _wk_taps(p1[0])
    w2k = _wk_s1(p2[0])
    wdk = jnp.transpose(pd[0].reshape(Co, Ci), (1, 0)).astype(BF16)
    Wp2 = ph.shape[4]
    out_shape = (B, Ho + 2, Wp2, Co) if pad_out else (B, Ho, Wo, Co)
    body = functools.partial(_block_ds_body, Ho=Ho, Wo=Wo, Ci=Ci, Co=Co,
                             tr=tr, pad_out=pad_out)
    return pl.pallas_call(
        body,
        out_shape=jax.ShapeDtypeStruct(out_shape, out_dtype),
        grid=(B // bb,),
        in_specs=[
            pl.BlockSpec((bb,) + ph.shape[1:], lambda i: (i, 0, 0, 0, 0, 0)),
            pl.BlockSpec(w1k.shape, lambda i: (0, 0, 0)),
            pl.BlockSpec(s1.shape, lambda i: (0, 0)),
            pl.BlockSpec(b1.shape, lambda i: (0, 0)),
            pl.BlockSpec(w2k.shape, lambda i: (0, 0)),
            pl.BlockSpec(s2.shape, lambda i: (0, 0)),
            pl.BlockSpec(b2.shape, lambda i: (0, 0)),
            pl.BlockSpec(wdk.shape, lambda i: (0, 0)),
            pl.BlockSpec(sd.shape, lambda i: (0, 0)),
            pl.BlockSpec(bd.shape, lambda i: (0, 0)),
        ],
        out_specs=pl.BlockSpec((bb,) + out_shape[1:], lambda i: (i, 0, 0, 0)),
        scratch_shapes=[pltpu.VMEM((bb, Ho + 2, Wp2, Co), BF16)],
        compiler_params=pltpu.CompilerParams(dimension_semantics=("parallel",)),
    )(ph, w1k, s1, b1, w2k, s2, b2, wdk, sd, bd)


def _stem_pool_body(p_ref, w_ref, s_ref, b_ref, o_ref, cp_ref):
    w = w_ref[...]
    s = s_ref[...]
    bsh = b_ref[...]
    o_ref[...] = jnp.zeros(o_ref.shape, o_ref.dtype)
    for c in range(8):
        lhs = p_ref[0, 14 * c:14 * c + 14, :, :].reshape(14 * 120, 48)
        zf = jnp.dot(lhs, w, preferred_element_type=F32).reshape(14, 120, 256)
        z = (zf[:, 0:112, 0:64] + zf[:, 1:113, 64:128] +
             zf[:, 2:114, 128:192] + zf[:, 3:115, 192:256])
        z = jnp.maximum(z * s + bsh, 0.0)
        rs = z.reshape(14, 56, 2, 64)
        m1 = jnp.maximum(rs[:, :, 0, :], rs[:, :, 1, :])
        odd = rs[:, :, 1, :]
        o1 = jnp.concatenate([jnp.zeros((14, 1, 64), F32), odd[:, 0:55, :]],
                             axis=1)
        cp_ref[7 * c:7 * c + 7] = jnp.maximum(m1, o1).reshape(7, 2, 56, 64)
    a = cp_ref[:, 0]
    bq = cp_ref[:, 1]
    bshift = jnp.concatenate([jnp.zeros((1, 56, 64), F32), bq[0:55]], axis=0)
    pooled = jnp.maximum(jnp.maximum(a, bq), bshift)
    o_ref[0, 1:57, 1:57, :] = pooled.astype(BF16)


def _mk_phases(x, hq):
    B, H, W, C = x.shape
    xq = jnp.pad(x, ((0, 0), (1, 2 * hq - H - 1), (1, 2 * hq - W - 1), (0, 0)))
    return xq.reshape(B, hq, 2, hq, 2, C).transpose(0, 2, 4, 1, 3, 5)


def kernel(x, stem_w, stem_gamma, stem_beta, stem_mean, stem_var, l1_0_conv1_w, l1_0_conv1_gamma, l1_0_conv1_beta, l1_0_conv1_mean, l1_0_conv1_var, l1_0_conv2_w, l1_0_conv2_gamma, l1_0_conv2_beta, l1_0_conv2_mean, l1_0_conv2_var, l1_1_conv1_w, l1_1_conv1_gamma, l1_1_conv1_beta, l1_1_conv1_mean, l1_1_conv1_var, l1_1_conv2_w, l1_1_conv2_gamma, l1_1_conv2_beta, l1_1_conv2_mean, l1_1_conv2_var, l2_0_conv1_w, l2_0_conv1_gamma, l2_0_conv1_beta, l2_0_conv1_mean, l2_0_conv1_var, l2_0_conv2_w, l2_0_conv2_gamma, l2_0_conv2_beta, l2_0_conv2_mean, l2_0_conv2_var, l2_0_ds_w, l2_0_ds_gamma, l2_0_ds_beta, l2_0_ds_mean, l2_0_ds_var, l2_1_conv1_w, l2_1_conv1_gamma, l2_1_conv1_beta, l2_1_conv1_mean, l2_1_conv1_var, l2_1_conv2_w, l2_1_conv2_gamma, l2_1_conv2_beta, l2_1_conv2_mean, l2_1_conv2_var, l3_0_conv1_w, l3_0_conv1_gamma, l3_0_conv1_beta, l3_0_conv1_mean, l3_0_conv1_var, l3_0_conv2_w, l3_0_conv2_gamma, l3_0_conv2_beta, l3_0_conv2_mean, l3_0_conv2_var, l3_0_ds_w, l3_0_ds_gamma, l3_0_ds_beta, l3_0_ds_mean, l3_0_ds_var, l3_1_conv1_w, l3_1_conv1_gamma, l3_1_conv1_beta, l3_1_conv1_mean, l3_1_conv1_var, l3_1_conv2_w, l3_1_conv2_gamma, l3_1_conv2_beta, l3_1_conv2_mean, l3_1_conv2_var):
    B = x.shape[0]

    xb = x.astype(BF16)
    xpad = jnp.pad(xb, ((0, 0), (0, 0), (3, 3), (3, 5)))
    xph = xpad.reshape(B, 3, 115, 2, 116, 2).transpose(0, 2, 4, 3, 5, 1)
    xph = jnp.pad(xph.reshape(B, 115, 116, 12),
                  ((0, 0), (0, 0), (0, 4), (0, 0)))
    patches = jnp.concatenate([xph[:, dy:dy + 112, :, :] for dy in range(4)],
                              axis=-1)
    wp = jnp.pad(stem_w, ((0, 0), (0, 0), (0, 1), (0, 1)))
    w_mat = wp.reshape(64, 3, 4, 2, 4, 2).transpose(2, 3, 5, 1, 4, 0)
    w_mat = w_mat.reshape(48, 256).astype(BF16)
    ss, sb = _fold_bn(stem_gamma, stem_beta, stem_mean, stem_var)
    x1 = pl.pallas_call(
        _stem_pool_body,
        out_shape=jax.ShapeDtypeStruct((B, 58, 64, 64), BF16),
        grid=(B,),
        in_specs=[
            pl.BlockSpec((1, 112, 120, 48), lambda i: (i, 0, 0, 0)),
            pl.BlockSpec((48, 256), lambda i: (0, 0)),
            pl.BlockSpec((1, 64), lambda i: (0, 0)),
            pl.BlockSpec((1, 64), lambda i: (0, 0)),
        ],
        out_specs=pl.BlockSpec((1, 58, 64, 64), lambda i: (i, 0, 0, 0)),
        scratch_shapes=[pltpu.VMEM((56, 2, 56, 64), F32)],
        compiler_params=pltpu.CompilerParams(dimension_semantics=("parallel",)),
    )(patches, w_mat, ss, sb)

    l10 = ((l1_0_conv1_w, l1_0_conv1_gamma, l1_0_conv1_beta, l1_0_conv1_mean, l1_0_conv1_var),
           (l1_0_conv2_w, l1_0_conv2_gamma, l1_0_conv2_beta, l1_0_conv2_mean, l1_0_conv2_var))
    l11 = ((l1_1_conv1_w, l1_1_conv1_gamma, l1_1_conv1_beta, l1_1_conv1_mean, l1_1_conv1_var),
           (l1_1_conv2_w, l1_1_conv2_gamma, l1_1_conv2_beta, l1_1_conv2_mean, l1_1_conv2_var))
    x1 = _block_s1(x1, *l10, H=56, W=56, C=64, tr=14, bb=1, pad_out=True,
                   out_dtype=BF16)
    q2 = _block_s1(x1, *l11, H=56, W=56, C=64, tr=14, bb=1, pad_out=False,
                   out_dtype=BF16, phase_out=True)

    l20 = ((l2_0_conv1_w, l2_0_conv1_gamma, l2_0_conv1_beta, l2_0_conv1_mean, l2_0_conv1_var),
           (l2_0_conv2_w, l2_0_conv2_gamma, l2_0_conv2_beta, l2_0_conv2_mean, l2_0_conv2_var),
           (l2_0_ds_w, l2_0_ds_gamma, l2_0_ds_beta, l2_0_ds_mean, l2_0_ds_var))
    l21 = ((l2_1_conv1_w, l2_1_conv1_gamma, l2_1_conv1_beta, l2_1_conv1_mean, l2_1_conv1_var),
           (l2_1_conv2_w, l2_1_conv2_gamma, l2_1_conv2_beta, l2_1_conv2_mean, l2_1_conv2_var))
    x2 = _block_ds(q2, *l20, Ho=28, Wo=28, Ci=64, Co=128, tr=14, bb=4,
                   pad_out=True, out_dtype=BF16)
    q3 = _block_s1(x2, *l21, H=28, W=28, C=128, tr=14, bb=4, pad_out=False,
                   out_dtype=BF16, phase_out=True)

    l30 = ((l3_0_conv1_w, l3_0_conv1_gamma, l3_0_conv1_beta, l3_0_conv1_mean, l3_0_conv1_var),
           (l3_0_conv2_w, l3_0_conv2_gamma, l3_0_conv2_beta, l3_0_conv2_mean, l3_0_conv2_var),
           (l3_0_ds_w, l3_0_ds_gamma, l3_0_ds_beta, l3_0_ds_mean, l3_0_ds_var))
    l31 = ((l3_1_conv1_w, l3_1_conv1_gamma, l3_1_conv1_beta, l3_1_conv1_mean, l3_1_conv1_var),
           (l3_1_conv2_w, l3_1_conv2_gamma, l3_1_conv2_beta, l3_1_conv2_mean, l3_1_conv2_var))
    x3 = _block_ds(q3, *l30, Ho=14, Wo=14, Ci=128, Co=256, tr=14, bb=8,
                   pad_out=True, out_dtype=BF16)
    x3 = _block_s1(x3, *l31, H=14, W=14, C=256, tr=14, bb=8, pad_out=False,
                   out_dtype=F32)

    return jnp.transpose(x3, (0, 3, 1, 2))
